# Initial kernel scaffold; baseline (speedup 1.0000x reference)
import jax
import jax.numpy as jnp
from jax.experimental import pallas as pl


def kernel(x):
    raise NotImplementedError("write your pallas kernel here")


# speedup vs baseline: 2.5452x; 2.5452x over previous
import functools

import jax
import jax.numpy as jnp
from jax import lax
from jax.experimental import pallas as pl
from jax.experimental.pallas import tpu as pltpu

_LANES = 128


def _keys_from_f32(e):
    u = lax.bitcast_convert_type(e, jnp.uint32)
    return jnp.where(u >= jnp.uint32(0x80000000), ~u, u | jnp.uint32(0x80000000))


def _f32_from_key(key):
    u = jnp.where(key >= jnp.uint32(0x80000000), key ^ jnp.uint32(0x80000000), ~key)
    return lax.bitcast_convert_type(u, jnp.float32)


def _energy_body(x_ref, e_ref):
    e_ref[...] = jnp.sum(x_ref[...], axis=1)


def _threshold_body(B, k, e_ref, t_ref):
    for b in range(B):
        key = _keys_from_f32(e_ref[b])

        def body(i, p):
            bit = jnp.uint32(0x80000000) >> i.astype(jnp.uint32)
            q = p | bit
            cnt = jnp.sum((key >= q).astype(jnp.int32))
            return jnp.where(cnt >= k, q, p)

        t = lax.fori_loop(0, 32, body, jnp.uint32(0))
        t_ref[b] = jnp.full((1, _LANES), _f32_from_key(t), dtype=jnp.float32)


def _mask_body(x_ref, e_ref, t_ref, o_ref):
    t = jnp.max(t_ref[...])
    keep = e_ref[...] >= t
    o_ref[...] = jnp.where(keep[:, None, :, :], x_ref[...], jnp.float32(0.0))


@jax.jit
def kernel(x):
    B, C, H, W = x.shape
    N = H * W
    k = N // 2
    assert N % _LANES == 0
    rows_total = N // _LANES

    n_chunks = 24 if rows_total % 24 == 0 else 1
    rows = rows_total // n_chunks

    xf = x.reshape(B, C, rows_total, _LANES)

    energy = pl.pallas_call(
        _energy_body,
        grid=(B, n_chunks),
        in_specs=[pl.BlockSpec((1, C, rows, _LANES), lambda b, j: (b, 0, j, 0))],
        out_specs=pl.BlockSpec((1, rows, _LANES), lambda b, j: (b, j, 0)),
        out_shape=jax.ShapeDtypeStruct((B, rows_total, _LANES), jnp.float32),
    )(xf)

    thresh = pl.pallas_call(
        functools.partial(_threshold_body, B, k),
        in_specs=[pl.BlockSpec((B, rows_total, _LANES), lambda: (0, 0, 0))],
        out_specs=pl.BlockSpec((B, 1, _LANES), lambda: (0, 0, 0)),
        out_shape=jax.ShapeDtypeStruct((B, 1, _LANES), jnp.float32),
    )(energy)

    out = pl.pallas_call(
        _mask_body,
        grid=(B, n_chunks),
        in_specs=[
            pl.BlockSpec((1, C, rows, _LANES), lambda b, j: (b, 0, j, 0)),
            pl.BlockSpec((1, rows, _LANES), lambda b, j: (b, j, 0)),
            pl.BlockSpec((1, 1, _LANES), lambda b, j: (b, 0, 0)),
        ],
        out_specs=pl.BlockSpec((1, C, rows, _LANES), lambda b, j: (b, 0, j, 0)),
        out_shape=jax.ShapeDtypeStruct((B, C, rows_total, _LANES), jnp.float32),
    )(xf, energy, thresh)

    return out.reshape(B, C, H, W)
